# baseline (device time: 72299 ns/iter reference)
import jax
import jax.numpy as jnp
from jax import lax
from jax.experimental import pallas as pl
from jax.experimental.pallas import tpu as pltpu

T = 2048
D = 1024
CH = 128
NCH = T // CH
PAD = 16


def kernel(x, dest):
    my_y = lax.axis_index("y")

    n0 = jnp.sum((dest == 0).astype(jnp.int32))
    n_keep = jnp.where(my_y == 0, n0, T - n0).astype(jnp.int32)
    m = (T - n_keep).astype(jnp.int32)
    keep_off = my_y * m
    recv_dst = (1 - my_y) * n_keep
    a = ((n_keep + 7) // 8) * 8
    nch = (m + CH - 1) // CH

    order2 = jnp.argsort(dest != my_y, stable=True)
    idx = jnp.arange(T + PAD, dtype=jnp.int32)
    j = jnp.where(idx < n_keep, idx, jnp.clip(idx - a + n_keep, 0, T - 1))
    xbuf = x[order2[jnp.clip(j, 0, T - 1)], :].astype(jnp.bfloat16)

    meta = jnp.stack([n_keep, m, a, keep_off, recv_dst, nch]).astype(jnp.int32)

    def body(meta_ref, xbuf_ref, out_ref, recv_ref, send_sems, recv_sems):
        n_keep_ = meta_ref[0]
        m_ = meta_ref[1]
        a_ = meta_ref[2]
        keep_off_ = meta_ref[3]
        recv_dst_ = meta_ref[4]
        nch_ = meta_ref[5]

        ax = lax.axis_index("x")
        ay = lax.axis_index("y")
        az = lax.axis_index("z")
        peer = (ax, 1 - ay, az)

        barrier = pltpu.get_barrier_semaphore()
        pl.semaphore_signal(
            barrier, inc=1, device_id=peer, device_id_type=pl.DeviceIdType.MESH
        )
        pl.semaphore_wait(barrier, 1)

        def cstart(i):
            tail = jnp.maximum(0, ((m_ + 7) // 8) * 8 - CH)
            s = jnp.where(i == nch_ - 1, tail, i * CH)
            return pl.multiple_of(s, 8)

        def mk_chunk(i):
            s = cstart(i)
            return pltpu.make_async_remote_copy(
                src_ref=xbuf_ref.at[pl.ds(pl.multiple_of(a_ + s, 8), CH), :],
                dst_ref=recv_ref.at[pl.ds(s, CH), :],
                send_sem=send_sems.at[i],
                recv_sem=recv_sems.at[i],
                device_id=peer,
                device_id_type=pl.DeviceIdType.MESH,
            )

        for i in range(NCH):
            rdma = mk_chunk(i)

            @pl.when(i < nch_)
            def _(rdma=rdma):
                rdma.start()

        out_ref[...] = pltpu.roll(xbuf_ref[0:T, :], keep_off_, 0)

        for i in range(NCH):
            rdma = mk_chunk(i)

            @pl.when(i < nch_)
            def _(rdma=rdma):
                rdma.wait_send()
                rdma.wait_recv()

        row_ids = lax.broadcasted_iota(jnp.int32, (T, 1), 0)
        mine = (row_ids >= keep_off_) & (row_ids < keep_off_ + n_keep_)
        out_ref[...] = jnp.where(
            mine, out_ref[...], pltpu.roll(recv_ref[...], recv_dst_, 0)
        )

    return pl.pallas_call(
        body,
        out_shape=jax.ShapeDtypeStruct((T, D), jnp.bfloat16),
        in_specs=[
            pl.BlockSpec(memory_space=pltpu.SMEM),
            pl.BlockSpec(memory_space=pltpu.VMEM),
        ],
        out_specs=pl.BlockSpec(memory_space=pltpu.VMEM),
        scratch_shapes=[
            pltpu.VMEM((T, D), jnp.bfloat16),
            pltpu.SemaphoreType.DMA((NCH,)),
            pltpu.SemaphoreType.DMA((NCH,)),
        ],
        compiler_params=pltpu.CompilerParams(collective_id=0),
    )(meta, xbuf)
